# Initial kernel scaffold; baseline (speedup 1.0000x reference)
import jax
import jax.numpy as jnp
from jax.experimental import pallas as pl


def kernel(x, edge_index, edge_weight, W1, b1, W2, b2):
    raise NotImplementedError("write your pallas kernel here")


# speedup vs baseline: 22.9923x; 22.9923x over previous
import functools

import jax
import jax.numpy as jnp
from jax import lax
from jax.experimental import pallas as pl
from jax.experimental.pallas import tpu as pltpu
from jax.experimental.pallas import tpu_sc as plsc

N = 10000
E = 320000
D = 128
NC = 2
NS = 16
NW = NC * NS
EPW = E // NW
WIN = 80
NWIN = EPW // WIN
NPAD = 10240
RPT = NPAD // NS
ZR = 32
BLK = 400
GRID = N // BLK

_mesh = plsc.VectorSubcoreMesh(core_axis_name="c", subcore_axis_name="s",
                               num_cores=NC, num_subcores=NS)


def _deg_body(col_hbm, ew_hbm, out_hbm, col_v, ew_v, zb, dacc, sem):
    c = lax.axis_index("c")
    s = lax.axis_index("s")
    wid = c * NS + s
    pltpu.sync_copy(col_hbm.at[wid], col_v)
    pltpu.sync_copy(ew_hbm.at[wid], ew_v)
    zv = jnp.zeros((16,), jnp.float32)

    def zset(i, carry):
        zb[pl.ds(i * 16, 16)] = zv
        return carry

    lax.fori_loop(0, RPT // 16, zset, 0)
    pltpu.sync_copy(zb, dacc.at[pl.ds(s * RPT, RPT)])
    plsc.subcore_barrier()

    def scat(w, carry):
        pltpu.async_copy(ew_v.at[w], dacc.at[col_v.at[w]], sem, add=True)
        return carry

    lax.fori_loop(0, NWIN, scat, 0)

    def drain(w, carry):
        pltpu.make_async_copy(ew_v.at[w], dacc.at[col_v.at[w]], sem).wait()
        return carry

    lax.fori_loop(0, NWIN, drain, 0)
    plsc.subcore_barrier()
    pltpu.sync_copy(dacc.at[pl.ds(s * RPT, RPT)],
                    out_hbm.at[c, pl.ds(s * RPT, RPT)])


_deg_call = pl.kernel(
    _deg_body,
    out_type=jax.ShapeDtypeStruct((NC, NPAD), jnp.float32),
    mesh=_mesh,
    scratch_types=[
        pltpu.VMEM((NWIN, WIN), jnp.int32),
        pltpu.VMEM((NWIN, WIN), jnp.float32),
        pltpu.VMEM((RPT,), jnp.float32),
        pltpu.VMEM_SHARED((NPAD,), jnp.float32),
        pltpu.SemaphoreType.DMA,
    ],
)


def _edge_body(t_hbm, row_hbm, cew_hbm, out_hbm,
               row_v, cbuf, gbuf, zb, acc, sem0, sem1, csem0, csem1):
    c = lax.axis_index("c")
    s = lax.axis_index("s")
    wid = c * NS + s
    pltpu.sync_copy(row_hbm.at[wid], row_v)
    pltpu.async_copy(t_hbm.at[row_v.at[0]], gbuf.at[0], sem0)
    pltpu.async_copy(cew_hbm.at[wid, 0], cbuf.at[0], csem0)

    zv = jnp.zeros((16,), jnp.float32)

    def zset(i, carry):
        for j in range(D // 16):
            zb[i, pl.ds(j * 16, 16)] = zv
        return carry

    lax.fori_loop(0, ZR, zset, 0)

    def zcopy(k, carry):
        pltpu.sync_copy(zb, acc.at[pl.ds(s * RPT + k * ZR, ZR)])
        return carry

    lax.fori_loop(0, RPT // ZR, zcopy, 0)
    plsc.subcore_barrier()

    def process(w, b):
        sem_cur = sem0 if b == 0 else sem1
        csem_cur = csem0 if b == 0 else csem1
        pltpu.make_async_copy(t_hbm.at[row_v.at[w]], gbuf.at[b],
                              sem_cur).wait()
        pltpu.make_async_copy(cew_hbm.at[wid, w], cbuf.at[b], csem_cur).wait()

        def scale(g, carry2):
            wvec = lax.bitcast_convert_type(cbuf[b, 1, pl.ds(g * 16, 16)],
                                            jnp.float32)
            for u in range(16):
                e = g * 16 + u
                wv = wvec[u]
                for k in range(D // 16):
                    sl = pl.ds(k * 16, 16)
                    gbuf[b, e, sl] = gbuf[b, e, sl] * wv
            return carry2

        lax.fori_loop(0, WIN // 16, scale, 0)
        pltpu.sync_copy(gbuf.at[b], acc.at[cbuf.at[b, 0]], add=True)

    def start(nxt, b_nxt):
        sem_nxt = sem0 if b_nxt == 0 else sem1
        csem_nxt = csem0 if b_nxt == 0 else csem1
        pltpu.async_copy(t_hbm.at[row_v.at[nxt]], gbuf.at[b_nxt], sem_nxt)
        pltpu.async_copy(cew_hbm.at[wid, nxt], cbuf.at[b_nxt], csem_nxt)

    def outer(i, carry):
        w0 = i * 2
        for b in range(2):
            w = w0 + b
            nxt = w + 1

            @pl.when(nxt < NWIN)
            def _():
                start(nxt, 1 - b)

            process(w, b)
        return carry

    lax.fori_loop(0, NWIN // 2, outer, 0)
    if NWIN % 2 == 1:
        process(NWIN - 1, 0)
    plsc.subcore_barrier()
    pltpu.sync_copy(acc.at[pl.ds(s * RPT, RPT)],
                    out_hbm.at[c, pl.ds(s * RPT, RPT)])


_edge_call = pl.kernel(
    _edge_body,
    out_type=jax.ShapeDtypeStruct((NC, NPAD, D), jnp.float32),
    mesh=_mesh,
    scratch_types=[
        pltpu.VMEM((NWIN, WIN), jnp.int32),
        pltpu.VMEM((2, 2, WIN), jnp.int32),
        pltpu.VMEM((2, WIN, D), jnp.float32),
        pltpu.VMEM((ZR, D), jnp.float32),
        pltpu.VMEM_SHARED((NPAD, D), jnp.float32),
        pltpu.SemaphoreType.DMA,
        pltpu.SemaphoreType.DMA,
        pltpu.SemaphoreType.DMA,
        pltpu.SemaphoreType.DMA,
    ],
)


def _mm_body(x_ref, w_ref, o_ref):
    o_ref[...] = jnp.dot(x_ref[...], w_ref[...],
                         preferred_element_type=jnp.float32)


_mm_call = pl.pallas_call(
    _mm_body,
    grid=(GRID,),
    in_specs=[
        pl.BlockSpec((BLK, D), lambda i: (i, 0)),
        pl.BlockSpec((D, D), lambda i: (0, 0)),
    ],
    out_specs=pl.BlockSpec((BLK, D), lambda i: (i, 0)),
    out_shape=jax.ShapeDtypeStruct((N, D), jnp.float32),
)


def _pre_body(degp_ref, z1_ref, dinv_ref, t1_ref):
    d = degp_ref[0] + degp_ref[1] + 1.0
    di = lax.rsqrt(d)
    dinv_ref[...] = di
    t1_ref[...] = di * z1_ref[...]


_pre_call = pl.pallas_call(
    _pre_body,
    grid=(GRID,),
    in_specs=[
        pl.BlockSpec((2, BLK, 1), lambda i: (0, i, 0)),
        pl.BlockSpec((BLK, D), lambda i: (i, 0)),
    ],
    out_specs=[
        pl.BlockSpec((BLK, 1), lambda i: (i, 0)),
        pl.BlockSpec((BLK, D), lambda i: (i, 0)),
    ],
    out_shape=[
        jax.ShapeDtypeStruct((N, 1), jnp.float32),
        jax.ShapeDtypeStruct((N, D), jnp.float32),
    ],
)


def _mid_body(sp_ref, t1_ref, dinv_ref, b1_ref, w2_ref, t2_ref):
    di = dinv_ref[...]
    u = di * (sp_ref[0] + sp_ref[1] + t1_ref[...]) + b1_ref[...]
    u = jnp.maximum(u, 0.0)
    t2_ref[...] = di * jnp.dot(u, w2_ref[...],
                               preferred_element_type=jnp.float32)


_mid_call = pl.pallas_call(
    _mid_body,
    grid=(GRID,),
    in_specs=[
        pl.BlockSpec((2, BLK, D), lambda i: (0, i, 0)),
        pl.BlockSpec((BLK, D), lambda i: (i, 0)),
        pl.BlockSpec((BLK, 1), lambda i: (i, 0)),
        pl.BlockSpec((1, D), lambda i: (0, 0)),
        pl.BlockSpec((D, D), lambda i: (0, 0)),
    ],
    out_specs=pl.BlockSpec((BLK, D), lambda i: (i, 0)),
    out_shape=jax.ShapeDtypeStruct((N, D), jnp.float32),
)


def _post_body(sp_ref, t2_ref, dinv_ref, b2_ref, o_ref):
    o_ref[...] = (dinv_ref[...] * (sp_ref[0] + sp_ref[1] + t2_ref[...])
                  + b2_ref[...])


_post_call = pl.pallas_call(
    _post_body,
    grid=(GRID,),
    in_specs=[
        pl.BlockSpec((2, BLK, D), lambda i: (0, i, 0)),
        pl.BlockSpec((BLK, D), lambda i: (i, 0)),
        pl.BlockSpec((BLK, 1), lambda i: (i, 0)),
        pl.BlockSpec((1, D), lambda i: (0, 0)),
    ],
    out_specs=pl.BlockSpec((BLK, D), lambda i: (i, 0)),
    out_shape=jax.ShapeDtypeStruct((N, D), jnp.float32),
)


def kernel(x, edge_index, edge_weight, W1, b1, W2, b2):
    row = edge_index[0].reshape(NW, NWIN, WIN)
    col = edge_index[1].reshape(NW, NWIN, WIN)
    ew = edge_weight.reshape(NW, NWIN, WIN)
    cew = jnp.stack([col, lax.bitcast_convert_type(ew, jnp.int32)], axis=2)

    degp = _deg_call(col, ew)
    z1 = _mm_call(x, W1)
    dinv, t1 = _pre_call(degp[:, :N, None], z1)
    s1 = _edge_call(t1, row, cew)
    t2 = _mid_call(s1[:, :N], t1, dinv, b1[None, :], W2)
    s2 = _edge_call(t2, row, cew)
    out = _post_call(s2[:, :N], t2, dinv, b2[None, :])
    return out
